# Initial kernel scaffold; baseline (speedup 1.0000x reference)
import jax
import jax.numpy as jnp
from jax.experimental import pallas as pl


def kernel(x, Wq, Wk, Wv, Wo, W1, W2, ln1, ln2, gate, ln_f):
    raise NotImplementedError("write your pallas kernel here")


# speedup vs baseline: 7.9284x; 7.9284x over previous
import functools

import jax
import jax.numpy as jnp
from jax.experimental import pallas as pl
from jax.experimental.pallas import tpu as pltpu

K_TOPK = 32
MEM_SET = (4, 5)


def _ln(x, scale):
    m = jnp.mean(x, axis=-1, keepdims=True)
    v = jnp.mean((x - m) ** 2, axis=-1, keepdims=True)
    return (x - m) * jax.lax.rsqrt(v + 1e-5) * scale


def _qkv_kernel(h_ref, ln1_ref, wq_ref, wk_ref, wv_ref, q_ref, k_ref, v_ref):
    a = _ln(h_ref[...], ln1_ref[...])
    q_ref[...] = jnp.dot(a, wq_ref[...], preferred_element_type=jnp.float32)
    k_ref[...] = jnp.dot(a, wk_ref[...], preferred_element_type=jnp.float32)
    v_ref[...] = jnp.dot(a, wv_ref[...], preferred_element_type=jnp.float32)


def _knn_readout(sim, vh, col, work_ref, sel_ref):
    s_len = sim.shape[-1]
    work_ref[...] = sim
    sel_ref[...] = jnp.zeros_like(sim)

    def body(_, tok):
        work = work_ref[...]
        m = jnp.max(work, axis=-1, keepdims=True)
        ism = work == m
        first = jnp.min(jnp.where(ism, col, s_len), axis=-1, keepdims=True)
        pick = col == first
        work_ref[...] = jnp.where(pick, -jnp.inf, work)
        sel_ref[...] = jnp.where(pick, 1.0, sel_ref[...])
        return tok

    jax.lax.fori_loop(0, K_TOPK, body, 0)
    mx = jnp.max(sim, axis=-1, keepdims=True)
    e = sel_ref[...] * jnp.exp(sim - mx)
    num = jnp.dot(e, vh, preferred_element_type=jnp.float32)
    return num / jnp.sum(e, axis=-1, keepdims=True)


def _attn_kernel(q_ref, k_ref, v_ref, g_ref, o_ref, work_ref, sel_ref,
                 *, mem, heads, dh, sc, s_len):
    qi = pl.program_id(0)
    scale = 1.0 / (dh ** 0.5)
    row = qi * sc + jax.lax.broadcasted_iota(jnp.int32, (sc, s_len), 0)
    col = jax.lax.broadcasted_iota(jnp.int32, (sc, s_len), 1)
    causal = row >= col
    for h in range(heads):
        sl = slice(h * dh, (h + 1) * dh)
        qh = q_ref[:, sl]
        kh = k_ref[:, sl]
        vh = v_ref[:, sl]
        sim = jax.lax.dot_general(
            qh, kh, (((1,), (1,)), ((), ())),
            preferred_element_type=jnp.float32) * scale
        loc = jnp.where(causal, sim, -1e9)
        mx = jnp.max(loc, axis=-1, keepdims=True)
        e = jnp.exp(loc - mx)
        out = jnp.dot(e, vh, preferred_element_type=jnp.float32)
        out = out / jnp.sum(e, axis=-1, keepdims=True)
        if mem:
            knn = _knn_readout(sim, vh, col, work_ref, sel_ref)
            g = jax.nn.sigmoid(g_ref[0, h])
            out = g * knn + (1.0 - g) * out
        o_ref[:, sl] = out


def _post_kernel(ao_ref, h_ref, wo_ref, w1_ref, w2_ref, ln2_ref, out_ref):
    h2 = h_ref[...] + jnp.dot(ao_ref[...], wo_ref[...],
                              preferred_element_type=jnp.float32)
    b = _ln(h2, ln2_ref[...])
    t = jax.nn.gelu(jnp.dot(b, w1_ref[...], preferred_element_type=jnp.float32))
    out_ref[...] = h2 + jnp.dot(t, w2_ref[...],
                                preferred_element_type=jnp.float32)


def _fln_kernel(h_ref, s_ref, o_ref):
    o_ref[...] = _ln(h_ref[...], s_ref[...])


def kernel(x, Wq, Wk, Wv, Wo, W1, W2, ln1, ln2, gate, ln_f):
    B, S, D = x.shape
    L, H = gate.shape
    DH = D // H
    DF = W1.shape[-1]
    SC = min(256, S)
    nq = S // SC

    row_spec = pl.BlockSpec((SC, D), lambda i: (i, 0))
    one_row = pl.BlockSpec((1, D), lambda i: (0, 0))
    full_dd = pl.BlockSpec((D, D), lambda i: (0, 0))
    full_sd = pl.BlockSpec((S, D), lambda i: (0, 0))

    qkv_call = pl.pallas_call(
        _qkv_kernel,
        grid=(nq,),
        in_specs=[row_spec, one_row, full_dd, full_dd, full_dd],
        out_specs=(row_spec, row_spec, row_spec),
        out_shape=tuple(jax.ShapeDtypeStruct((S, D), jnp.float32)
                        for _ in range(3)),
    )

    def attn_call(mem):
        return pl.pallas_call(
            functools.partial(_attn_kernel, mem=mem, heads=H, dh=DH,
                              sc=SC, s_len=S),
            grid=(nq,),
            in_specs=[row_spec, full_sd, full_sd,
                      pl.BlockSpec((1, H), lambda i: (0, 0))],
            out_specs=row_spec,
            out_shape=jax.ShapeDtypeStruct((S, D), jnp.float32),
            scratch_shapes=[pltpu.VMEM((SC, S), jnp.float32),
                            pltpu.VMEM((SC, S), jnp.float32)],
        )

    attn_plain = attn_call(False)
    attn_mem = attn_call(True)

    post_call = pl.pallas_call(
        _post_kernel,
        grid=(nq,),
        in_specs=[row_spec, row_spec, full_dd,
                  pl.BlockSpec((D, DF), lambda i: (0, 0)),
                  pl.BlockSpec((DF, D), lambda i: (0, 0)),
                  one_row],
        out_specs=row_spec,
        out_shape=jax.ShapeDtypeStruct((S, D), jnp.float32),
    )

    fln_call = pl.pallas_call(
        _fln_kernel,
        grid=(nq,),
        in_specs=[row_spec, one_row],
        out_specs=row_spec,
        out_shape=jax.ShapeDtypeStruct((S, D), jnp.float32),
    )

    h = x[0]
    for i in range(L):
        q, k, v = qkv_call(h, ln1[i][None], Wq[i], Wk[i], Wv[i])
        attn = attn_mem if i in MEM_SET else attn_plain
        ao = attn(q, k, v, gate[i][None])
        h = post_call(ao, h, Wo[i], W1[i], W2[i], ln2[i][None])
    out = fln_call(h, ln_f[None])
    return out[None]


# speedup vs baseline: 9.3725x; 1.1821x over previous
import functools

import jax
import jax.numpy as jnp
from jax.experimental import pallas as pl
from jax.experimental.pallas import tpu as pltpu

K_TOPK = 32
MEM_SET = (4, 5)


def _ln(x, scale):
    m = jnp.mean(x, axis=-1, keepdims=True)
    v = jnp.mean((x - m) ** 2, axis=-1, keepdims=True)
    return (x - m) * jax.lax.rsqrt(v + 1e-5) * scale


def _qkv_kernel(h_ref, ln1_ref, wq_ref, wk_ref, wv_ref, q_ref, k_ref, v_ref):
    a = _ln(h_ref[...], ln1_ref[...])
    q_ref[...] = jnp.dot(a, wq_ref[...], preferred_element_type=jnp.float32)
    k_ref[...] = jnp.dot(a, wk_ref[...], preferred_element_type=jnp.float32)
    v_ref[...] = jnp.dot(a, wv_ref[...], preferred_element_type=jnp.float32)


def _knn_readout(sim, vh, col, work_ref):
    s_len = sim.shape[-1]
    work_ref[...] = sim

    def body(_, tok):
        work = work_ref[...]
        m = jnp.max(work, axis=-1, keepdims=True)
        ism = work == m
        first = jnp.min(jnp.where(ism, col, s_len), axis=-1, keepdims=True)
        work_ref[...] = jnp.where(col == first, -jnp.inf, work)
        return tok

    jax.lax.fori_loop(0, K_TOPK, body, 0)
    mx = jnp.max(sim, axis=-1, keepdims=True)
    e = jnp.where(work_ref[...] == -jnp.inf, jnp.exp(sim - mx), 0.0)
    num = jnp.dot(e, vh, preferred_element_type=jnp.float32)
    return num / jnp.sum(e, axis=-1, keepdims=True)


def _attn_kernel(q_ref, k_ref, v_ref, g_ref, o_ref, work_ref,
                 *, mem, heads, dh, sc, s_len):
    qi = pl.program_id(0)
    scale = 1.0 / (dh ** 0.5)
    row = qi * sc + jax.lax.broadcasted_iota(jnp.int32, (sc, s_len), 0)
    col = jax.lax.broadcasted_iota(jnp.int32, (sc, s_len), 1)
    causal = row >= col
    for h in range(heads):
        sl = slice(h * dh, (h + 1) * dh)
        qh = q_ref[:, sl]
        kh = k_ref[:, sl]
        vh = v_ref[:, sl]
        sim = jax.lax.dot_general(
            qh, kh, (((1,), (1,)), ((), ())),
            preferred_element_type=jnp.float32) * scale
        loc = jnp.where(causal, sim, -1e9)
        mx = jnp.max(loc, axis=-1, keepdims=True)
        e = jnp.exp(loc - mx)
        out = jnp.dot(e, vh, preferred_element_type=jnp.float32)
        out = out / jnp.sum(e, axis=-1, keepdims=True)
        if mem:
            knn = _knn_readout(sim, vh, col, work_ref)
            g = jax.nn.sigmoid(g_ref[0, h])
            out = g * knn + (1.0 - g) * out
        o_ref[:, sl] = out


def _post_kernel(ao_ref, h_ref, wo_ref, w1_ref, w2_ref, ln2_ref, out_ref):
    h2 = h_ref[...] + jnp.dot(ao_ref[...], wo_ref[...],
                              preferred_element_type=jnp.float32)
    b = _ln(h2, ln2_ref[...])
    t = jax.nn.gelu(jnp.dot(b, w1_ref[...], preferred_element_type=jnp.float32))
    out_ref[...] = h2 + jnp.dot(t, w2_ref[...],
                                preferred_element_type=jnp.float32)


def _fln_kernel(h_ref, s_ref, o_ref):
    o_ref[...] = _ln(h_ref[...], s_ref[...])


def kernel(x, Wq, Wk, Wv, Wo, W1, W2, ln1, ln2, gate, ln_f):
    B, S, D = x.shape
    L, H = gate.shape
    DH = D // H
    DF = W1.shape[-1]
    SC = min(256, S)
    nq = S // SC

    row_spec = pl.BlockSpec((SC, D), lambda i: (i, 0))
    one_row = pl.BlockSpec((1, D), lambda i: (0, 0))
    full_dd = pl.BlockSpec((D, D), lambda i: (0, 0))
    full_sd = pl.BlockSpec((S, D), lambda i: (0, 0))

    qkv_call = pl.pallas_call(
        _qkv_kernel,
        grid=(nq,),
        in_specs=[row_spec, one_row, full_dd, full_dd, full_dd],
        out_specs=(row_spec, row_spec, row_spec),
        out_shape=tuple(jax.ShapeDtypeStruct((S, D), jnp.float32)
                        for _ in range(3)),
    )

    def attn_call(mem):
        return pl.pallas_call(
            functools.partial(_attn_kernel, mem=mem, heads=H, dh=DH,
                              sc=SC, s_len=S),
            grid=(nq,),
            in_specs=[row_spec, full_sd, full_sd,
                      pl.BlockSpec((1, H), lambda i: (0, 0))],
            out_specs=row_spec,
            out_shape=jax.ShapeDtypeStruct((S, D), jnp.float32),
            scratch_shapes=[pltpu.VMEM((SC, S), jnp.float32)],
        )

    attn_plain = attn_call(False)
    attn_mem = attn_call(True)

    post_call = pl.pallas_call(
        _post_kernel,
        grid=(nq,),
        in_specs=[row_spec, row_spec, full_dd,
                  pl.BlockSpec((D, DF), lambda i: (0, 0)),
                  pl.BlockSpec((DF, D), lambda i: (0, 0)),
                  one_row],
        out_specs=row_spec,
        out_shape=jax.ShapeDtypeStruct((S, D), jnp.float32),
    )

    fln_call = pl.pallas_call(
        _fln_kernel,
        grid=(nq,),
        in_specs=[row_spec, one_row],
        out_specs=row_spec,
        out_shape=jax.ShapeDtypeStruct((S, D), jnp.float32),
    )

    h = x[0]
    for i in range(L):
        q, k, v = qkv_call(h, ln1[i][None], Wq[i], Wk[i], Wv[i])
        attn = attn_mem if i in MEM_SET else attn_plain
        ao = attn(q, k, v, gate[i][None])
        h = post_call(ao, h, Wo[i], W1[i], W2[i], ln2[i][None])
    out = fln_call(h, ln_f[None])
    return out[None]


# speedup vs baseline: 17.6443x; 1.8826x over previous
import functools

import jax
import jax.numpy as jnp
from jax.experimental import pallas as pl
from jax.experimental.pallas import tpu as pltpu

K_TOPK = 32
MEM_SET = (4, 5)


def _ln(x, scale):
    m = jnp.mean(x, axis=-1, keepdims=True)
    v = jnp.mean((x - m) ** 2, axis=-1, keepdims=True)
    return (x - m) * jax.lax.rsqrt(v + 1e-5) * scale


def _qkv_kernel(h_ref, ln1_ref, wq_ref, wk_ref, wv_ref, q_ref, k_ref, v_ref):
    a = _ln(h_ref[...], ln1_ref[...])
    q_ref[...] = jnp.dot(a, wq_ref[...], preferred_element_type=jnp.float32)
    k_ref[...] = jnp.dot(a, wk_ref[...], preferred_element_type=jnp.float32)
    v_ref[...] = jnp.dot(a, wv_ref[...], preferred_element_type=jnp.float32)


def _knn_readout(sim, vh, work_ref):
    mx = jnp.max(sim, axis=-1, keepdims=True)
    work_ref[...] = jnp.where(sim == mx, -jnp.inf, sim)

    def body(_, tok):
        work = work_ref[...]
        m = jnp.max(work, axis=-1, keepdims=True)
        work_ref[...] = jnp.where(work == m, -jnp.inf, work)
        return tok

    jax.lax.fori_loop(1, K_TOPK, body, 0)
    e = jnp.where(work_ref[...] == -jnp.inf, jnp.exp(sim - mx), 0.0)
    num = jnp.dot(e, vh, preferred_element_type=jnp.float32)
    return num / jnp.sum(e, axis=-1, keepdims=True)


def _attn_kernel(q_ref, k_ref, v_ref, g_ref, o_ref, work_ref,
                 *, mem, heads, dh, sc, s_len):
    qi = pl.program_id(0)
    scale = 1.0 / (dh ** 0.5)
    row = qi * sc + jax.lax.broadcasted_iota(jnp.int32, (sc, s_len), 0)
    col = jax.lax.broadcasted_iota(jnp.int32, (sc, s_len), 1)
    causal = row >= col
    for h in range(heads):
        sl = slice(h * dh, (h + 1) * dh)
        qh = q_ref[:, sl]
        kh = k_ref[:, sl]
        vh = v_ref[:, sl]
        sim = jax.lax.dot_general(
            qh, kh, (((1,), (1,)), ((), ())),
            preferred_element_type=jnp.float32) * scale
        loc = jnp.where(causal, sim, -1e9)
        mx = jnp.max(loc, axis=-1, keepdims=True)
        e = jnp.exp(loc - mx)
        out = jnp.dot(e, vh, preferred_element_type=jnp.float32)
        out = out / jnp.sum(e, axis=-1, keepdims=True)
        if mem:
            knn = _knn_readout(sim, vh, work_ref)
            g = jax.nn.sigmoid(g_ref[0, h])
            out = g * knn + (1.0 - g) * out
        o_ref[:, sl] = out


def _post_kernel(ao_ref, h_ref, wo_ref, w1_ref, w2_ref, ln2_ref, out_ref):
    h2 = h_ref[...] + jnp.dot(ao_ref[...], wo_ref[...],
                              preferred_element_type=jnp.float32)
    b = _ln(h2, ln2_ref[...])
    t = jax.nn.gelu(jnp.dot(b, w1_ref[...], preferred_element_type=jnp.float32))
    out_ref[...] = h2 + jnp.dot(t, w2_ref[...],
                                preferred_element_type=jnp.float32)


def _fln_kernel(h_ref, s_ref, o_ref):
    o_ref[...] = _ln(h_ref[...], s_ref[...])


def kernel(x, Wq, Wk, Wv, Wo, W1, W2, ln1, ln2, gate, ln_f):
    B, S, D = x.shape
    L, H = gate.shape
    DH = D // H
    DF = W1.shape[-1]
    SC = min(256, S)
    nq = S // SC

    row_spec = pl.BlockSpec((SC, D), lambda i: (i, 0))
    one_row = pl.BlockSpec((1, D), lambda i: (0, 0))
    full_dd = pl.BlockSpec((D, D), lambda i: (0, 0))
    full_sd = pl.BlockSpec((S, D), lambda i: (0, 0))

    qkv_call = pl.pallas_call(
        _qkv_kernel,
        grid=(nq,),
        in_specs=[row_spec, one_row, full_dd, full_dd, full_dd],
        out_specs=(row_spec, row_spec, row_spec),
        out_shape=tuple(jax.ShapeDtypeStruct((S, D), jnp.float32)
                        for _ in range(3)),
    )

    def attn_call(mem):
        return pl.pallas_call(
            functools.partial(_attn_kernel, mem=mem, heads=H, dh=DH,
                              sc=SC, s_len=S),
            grid=(nq,),
            in_specs=[row_spec, full_sd, full_sd,
                      pl.BlockSpec((1, H), lambda i: (0, 0))],
            out_specs=row_spec,
            out_shape=jax.ShapeDtypeStruct((S, D), jnp.float32),
            scratch_shapes=[pltpu.VMEM((SC, S), jnp.float32)],
        )

    attn_plain = attn_call(False)
    attn_mem = attn_call(True)

    post_call = pl.pallas_call(
        _post_kernel,
        grid=(nq,),
        in_specs=[row_spec, row_spec, full_dd,
                  pl.BlockSpec((D, DF), lambda i: (0, 0)),
                  pl.BlockSpec((DF, D), lambda i: (0, 0)),
                  one_row],
        out_specs=row_spec,
        out_shape=jax.ShapeDtypeStruct((S, D), jnp.float32),
    )

    fln_call = pl.pallas_call(
        _fln_kernel,
        grid=(nq,),
        in_specs=[row_spec, one_row],
        out_specs=row_spec,
        out_shape=jax.ShapeDtypeStruct((S, D), jnp.float32),
    )

    h = x[0]
    for i in range(L):
        q, k, v = qkv_call(h, ln1[i][None], Wq[i], Wk[i], Wv[i])
        attn = attn_mem if i in MEM_SET else attn_plain
        ao = attn(q, k, v, gate[i][None])
        h = post_call(ao, h, Wo[i], W1[i], W2[i], ln2[i][None])
    out = fln_call(h, ln_f[None])
    return out[None]
